# baseline (speedup 1.0000x reference)
import functools

import jax
import jax.numpy as jnp
from jax import lax
from jax.experimental import pallas as pl
from jax.experimental.pallas import tpu as pltpu, tpu_sc as plsc

N = 10000
D = 128
E = 320000

NC = 2
NS = 16
NW = NC * NS
CL = 128
CH = 80
HF = CH // 2
EPAD = NW * CH * CL
NP = 10240
RPS = NP // NS
RCH = RPS // CL

def _zero_block(ref, rows):
    cols = ref.shape[1] // 16

    def body(i, _):
        for k in range(cols):
            ref[i, pl.ds(k * 16, 16)] = jnp.zeros((16,), jnp.float32)
        return 0

    lax.fori_loop(0, rows, body, 0)


def _fill_rowids(idx2, base):
    def body(i, _):
        r = i // 8
        col = (i % 8) * 16
        idx2[r, pl.ds(col, 16)] = base + i * 16 + lax.iota(jnp.int32, 16)
        return 0

    lax.fori_loop(0, idx2.shape[0] * 8, body, 0)


def _deg_body(dst_hbm, out_hbm, idx_v, idx2_v, ones_v, acc_sh):
    c = lax.axis_index("c")
    s = lax.axis_index("s")
    w = c * NS + s

    _zero_block(ones_v, CL)
    _fill_rowids(idx2_v, s * RPS)

    for r in range(RCH):
        pltpu.sync_copy(ones_v, acc_sh.at[idx2_v.at[r]])
    plsc.subcore_barrier()

    one0 = jnp.where(lax.iota(jnp.int32, 16) == 0, 1.0, 0.0).astype(jnp.float32)

    def fill_ones(i, _):
        ones_v[i, pl.ds(0, 16)] = one0
        return 0

    lax.fori_loop(0, CL, fill_ones, 0)

    pltpu.sync_copy(dst_hbm.at[w], idx_v)

    def body(j, _):
        pltpu.sync_copy(ones_v, acc_sh.at[idx_v.at[j]], add=True)
        return 0

    lax.fori_loop(0, CH, body, 0)
    plsc.subcore_barrier()

    for r in range(RCH):
        pltpu.sync_copy(acc_sh.at[idx2_v.at[r]], ones_v)
        pltpu.sync_copy(ones_v, out_hbm.at[c, pl.ds(s * RPS + r * CL, CL)])


def _agg_body(g_hbm, src_hbm, dst_hbm, out_hbm,
              src_v, dst_v, idx2_v, rows0_v, rows1_v, sem_g, sem_s, acc_sh):
    c = lax.axis_index("c")
    s = lax.axis_index("s")
    w = c * NS + s

    _zero_block(rows0_v, CL)
    _fill_rowids(idx2_v, s * RPS)
    for r in range(RCH):
        pltpu.sync_copy(rows0_v, acc_sh.at[idx2_v.at[r]])
    plsc.subcore_barrier()

    for h in range(2):
        pltpu.sync_copy(src_hbm.at[w, pl.ds(h * HF, HF)], src_v)
        pltpu.sync_copy(dst_hbm.at[w, pl.ds(h * HF, HF)], dst_v)

        def body(j, _):
            pltpu.sync_copy(g_hbm.at[src_v.at[j]], rows0_v)
            pltpu.sync_copy(rows0_v, acc_sh.at[dst_v.at[j]], add=True)
            return 0

        lax.fori_loop(0, HF, body, 0)

    plsc.subcore_barrier()

    for r in range(RCH):
        pltpu.sync_copy(acc_sh.at[idx2_v.at[r]], rows0_v)
        pltpu.sync_copy(rows0_v, out_hbm.at[c, pl.ds(s * RPS + r * CL, CL)])


@functools.cache
def _sc_kernels():
    mesh = plsc.VectorSubcoreMesh(core_axis_name="c", subcore_axis_name="s")
    deg_k = pl.kernel(
        _deg_body,
        out_type=jax.ShapeDtypeStruct((NC, NP, D), jnp.float32),
        mesh=mesh,
        scratch_types=[
            pltpu.VMEM((CH, CL), jnp.int32),
            pltpu.VMEM((RCH, CL), jnp.int32),
            pltpu.VMEM((CL, D), jnp.float32),
            pltpu.VMEM_SHARED((NP, D), jnp.float32),
        ],
    )
    agg_k = pl.kernel(
        _agg_body,
        out_type=jax.ShapeDtypeStruct((NC, NP, D), jnp.float32),
        mesh=mesh,
        scratch_types=[
            pltpu.VMEM((HF, CL), jnp.int32),
            pltpu.VMEM((HF, CL), jnp.int32),
            pltpu.VMEM((RCH, CL), jnp.int32),
            pltpu.VMEM((CL, D), jnp.float32),
            pltpu.VMEM((CL, D), jnp.float32),
            pltpu.SemaphoreType.DMA,
            pltpu.SemaphoreType.DMA,
            pltpu.VMEM_SHARED((NP, D), jnp.float32),
        ],
    )
    return deg_k, agg_k


def _dis_block(degp):
    deg = degp[0, :, 0] + degp[1, :, 0]
    return lax.rsqrt(deg + 1.0)


def _dense1_body(degp_ref, x_ref, w_ref, g_ref):
    dis = _dis_block(degp_ref[...])
    h = jnp.dot(x_ref[...], w_ref[...], preferred_element_type=jnp.float32)
    g_ref[...] = h * dis[:, None]


def _dense2_body(degp_ref, aggp_ref, g_ref, w_ref, b_ref, o_ref):
    dis = _dis_block(degp_ref[...])
    agg = aggp_ref[0] + aggp_ref[1] + g_ref[...]
    act = jnp.maximum(agg * dis[:, None] + b_ref[...], 0.0)
    h = jnp.dot(act, w_ref[...], preferred_element_type=jnp.float32)
    o_ref[...] = h * dis[:, None]


def _final_body(degp_ref, aggp_ref, g_ref, b_ref, o_ref):
    dis = _dis_block(degp_ref[...])
    agg = aggp_ref[0] + aggp_ref[1] + g_ref[...]
    o_ref[...] = agg * dis[:, None] + b_ref[...]


_BR = 1024
_BRF = 1000

_dense1 = pl.pallas_call(
    _dense1_body,
    grid=(NP // _BR,),
    in_specs=[
        pl.BlockSpec((NC, _BR, D), lambda i: (0, i, 0)),
        pl.BlockSpec((_BR, D), lambda i: (i, 0)),
        pl.BlockSpec((D, D), lambda i: (0, 0)),
    ],
    out_specs=pl.BlockSpec((_BR, D), lambda i: (i, 0)),
    out_shape=jax.ShapeDtypeStruct((NP, D), jnp.float32),
)

_dense2 = pl.pallas_call(
    _dense2_body,
    grid=(NP // _BR,),
    in_specs=[
        pl.BlockSpec((NC, _BR, D), lambda i: (0, i, 0)),
        pl.BlockSpec((NC, _BR, D), lambda i: (0, i, 0)),
        pl.BlockSpec((_BR, D), lambda i: (i, 0)),
        pl.BlockSpec((D, D), lambda i: (0, 0)),
        pl.BlockSpec((1, D), lambda i: (0, 0)),
    ],
    out_specs=pl.BlockSpec((_BR, D), lambda i: (i, 0)),
    out_shape=jax.ShapeDtypeStruct((NP, D), jnp.float32),
)

_final = pl.pallas_call(
    _final_body,
    grid=(N // _BRF,),
    in_specs=[
        pl.BlockSpec((NC, _BRF, D), lambda i: (0, i, 0)),
        pl.BlockSpec((NC, _BRF, D), lambda i: (0, i, 0)),
        pl.BlockSpec((_BRF, D), lambda i: (i, 0)),
        pl.BlockSpec((1, D), lambda i: (0, 0)),
    ],
    out_specs=pl.BlockSpec((_BRF, D), lambda i: (i, 0)),
    out_shape=jax.ShapeDtypeStruct((N, D), jnp.float32),
)


def kernel(x, edge_index, W1, b1, W2, b2):
    src = edge_index[0].astype(jnp.int32)
    dst = edge_index[1].astype(jnp.int32)
    pad = EPAD - E
    pad_dst = N + (jnp.arange(pad, dtype=jnp.int32) % (NP - N))
    src_p = jnp.concatenate([src, jnp.zeros((pad,), jnp.int32)]).reshape(NW, CH, CL)
    dst_p = jnp.concatenate([dst, pad_dst]).reshape(NW, CH, CL)
    x_p = jnp.pad(x, ((0, NP - N), (0, 0)))
    b1r = b1.reshape(1, D)
    b2r = b2.reshape(1, D)

    deg_k, agg_k = _sc_kernels()
    degp = deg_k(dst_p)
    g1 = _dense1(degp, x_p, W1)
    agg1 = agg_k(g1, src_p, dst_p)
    g2 = _dense2(degp, agg1, g1, W2, b1r)
    agg2 = agg_k(g2, src_p, dst_p)
    return _final(degp, agg2, g2, b2r)


# speedup vs baseline: 1.5046x; 1.5046x over previous
import functools

import jax
import jax.numpy as jnp
from jax import lax
from jax.experimental import pallas as pl
from jax.experimental.pallas import tpu as pltpu, tpu_sc as plsc

N = 10000
D = 128
E = 320000

NC = 2
NS = 16
NW = NC * NS
CL = 128
CH = 79
EPAD = NW * CH * CL
NP = 10240
RPS = NP // NS
RCH = RPS // CL

def _zero_block(ref, rows):
    cols = ref.shape[1] // 16

    def body(i, _):
        for k in range(cols):
            ref[i, pl.ds(k * 16, 16)] = jnp.zeros((16,), jnp.float32)
        return 0

    lax.fori_loop(0, rows, body, 0)


def _fill_rowids(idx2, base):
    def body(i, _):
        r = i // 8
        col = (i % 8) * 16
        idx2[r, pl.ds(col, 16)] = base + i * 16 + lax.iota(jnp.int32, 16)
        return 0

    lax.fori_loop(0, idx2.shape[0] * 8, body, 0)


def _deg_body(dst_hbm, out_hbm, idx_v, idx2_v, ones_v, acc_sh):
    c = lax.axis_index("c")
    s = lax.axis_index("s")
    w = c * NS + s

    _zero_block(ones_v, CL)
    _fill_rowids(idx2_v, s * RPS)

    for r in range(RCH):
        pltpu.sync_copy(ones_v, acc_sh.at[idx2_v.at[r]])
    plsc.subcore_barrier()

    one0 = jnp.where(lax.iota(jnp.int32, 16) == 0, 1.0, 0.0).astype(jnp.float32)

    def fill_ones(i, _):
        ones_v[i, pl.ds(0, 16)] = one0
        return 0

    lax.fori_loop(0, CL, fill_ones, 0)

    pltpu.sync_copy(dst_hbm.at[w], idx_v)

    def body(j, _):
        pltpu.sync_copy(ones_v, acc_sh.at[idx_v.at[j]], add=True)
        return 0

    lax.fori_loop(0, CH, body, 0)
    plsc.subcore_barrier()

    for r in range(RCH):
        pltpu.sync_copy(acc_sh.at[idx2_v.at[r]], ones_v)
        pltpu.sync_copy(ones_v, out_hbm.at[c, pl.ds(s * RPS + r * CL, CL)])


def _agg_body(g_hbm, src_hbm, dst_hbm, out_hbm,
              src_v, dst_v, idx2_v, rows0_v, acc_sh):
    c = lax.axis_index("c")
    s = lax.axis_index("s")
    w = c * NS + s

    _zero_block(rows0_v, CL)
    _fill_rowids(idx2_v, s * RPS)
    for r in range(RCH):
        pltpu.sync_copy(rows0_v, acc_sh.at[idx2_v.at[r]])
    plsc.subcore_barrier()

    pltpu.sync_copy(src_hbm.at[w], src_v)
    pltpu.sync_copy(dst_hbm.at[w], dst_v)

    def body(j, _):
        pltpu.sync_copy(g_hbm.at[src_v.at[j]], rows0_v)
        pltpu.sync_copy(rows0_v, acc_sh.at[dst_v.at[j]], add=True)
        return 0

    lax.fori_loop(0, CH, body, 0)

    plsc.subcore_barrier()

    for r in range(RCH):
        pltpu.sync_copy(acc_sh.at[idx2_v.at[r]], rows0_v)
        pltpu.sync_copy(rows0_v, out_hbm.at[c, pl.ds(s * RPS + r * CL, CL)])


@functools.cache
def _sc_kernels():
    mesh = plsc.VectorSubcoreMesh(core_axis_name="c", subcore_axis_name="s")
    deg_k = pl.kernel(
        _deg_body,
        out_type=jax.ShapeDtypeStruct((NC, NP, D), jnp.float32),
        mesh=mesh,
        scratch_types=[
            pltpu.VMEM((CH, CL), jnp.int32),
            pltpu.VMEM((RCH, CL), jnp.int32),
            pltpu.VMEM((CL, D), jnp.float32),
            pltpu.VMEM_SHARED((NP, D), jnp.float32),
        ],
    )
    agg_k = pl.kernel(
        _agg_body,
        out_type=jax.ShapeDtypeStruct((NC, NP, D), jnp.float32),
        mesh=mesh,
        scratch_types=[
            pltpu.VMEM((CH, CL), jnp.int32),
            pltpu.VMEM((CH, CL), jnp.int32),
            pltpu.VMEM((RCH, CL), jnp.int32),
            pltpu.VMEM((CL, D), jnp.float32),
            pltpu.VMEM_SHARED((NP, D), jnp.float32),
        ],
    )
    return deg_k, agg_k


def _dis_block(degp):
    deg = degp[0, :, 0] + degp[1, :, 0]
    return lax.rsqrt(deg + 1.0)


def _dense1_body(degp_ref, x_ref, w_ref, g_ref):
    dis = _dis_block(degp_ref[...])
    h = jnp.dot(x_ref[...], w_ref[...], preferred_element_type=jnp.float32)
    g_ref[...] = h * dis[:, None]


def _dense2_body(degp_ref, aggp_ref, g_ref, w_ref, b_ref, o_ref):
    dis = _dis_block(degp_ref[...])
    agg = aggp_ref[0] + aggp_ref[1] + g_ref[...]
    act = jnp.maximum(agg * dis[:, None] + b_ref[...], 0.0)
    h = jnp.dot(act, w_ref[...], preferred_element_type=jnp.float32)
    o_ref[...] = h * dis[:, None]


def _final_body(degp_ref, aggp_ref, g_ref, b_ref, o_ref):
    dis = _dis_block(degp_ref[...])
    agg = aggp_ref[0] + aggp_ref[1] + g_ref[...]
    o_ref[...] = agg * dis[:, None] + b_ref[...]


_BR = 1024
_BRF = 1000

_dense1 = pl.pallas_call(
    _dense1_body,
    grid=(NP // _BR,),
    in_specs=[
        pl.BlockSpec((NC, _BR, D), lambda i: (0, i, 0)),
        pl.BlockSpec((_BR, D), lambda i: (i, 0)),
        pl.BlockSpec((D, D), lambda i: (0, 0)),
    ],
    out_specs=pl.BlockSpec((_BR, D), lambda i: (i, 0)),
    out_shape=jax.ShapeDtypeStruct((NP, D), jnp.float32),
)

_dense2 = pl.pallas_call(
    _dense2_body,
    grid=(NP // _BR,),
    in_specs=[
        pl.BlockSpec((NC, _BR, D), lambda i: (0, i, 0)),
        pl.BlockSpec((NC, _BR, D), lambda i: (0, i, 0)),
        pl.BlockSpec((_BR, D), lambda i: (i, 0)),
        pl.BlockSpec((D, D), lambda i: (0, 0)),
        pl.BlockSpec((1, D), lambda i: (0, 0)),
    ],
    out_specs=pl.BlockSpec((_BR, D), lambda i: (i, 0)),
    out_shape=jax.ShapeDtypeStruct((NP, D), jnp.float32),
)

_final = pl.pallas_call(
    _final_body,
    grid=(N // _BRF,),
    in_specs=[
        pl.BlockSpec((NC, _BRF, D), lambda i: (0, i, 0)),
        pl.BlockSpec((NC, _BRF, D), lambda i: (0, i, 0)),
        pl.BlockSpec((_BRF, D), lambda i: (i, 0)),
        pl.BlockSpec((1, D), lambda i: (0, 0)),
    ],
    out_specs=pl.BlockSpec((_BRF, D), lambda i: (i, 0)),
    out_shape=jax.ShapeDtypeStruct((N, D), jnp.float32),
)


def kernel(x, edge_index, W1, b1, W2, b2):
    src = edge_index[0].astype(jnp.int32)
    dst = edge_index[1].astype(jnp.int32)
    pad = EPAD - E
    pad_dst = N + (jnp.arange(pad, dtype=jnp.int32) % (NP - N))
    src_p = jnp.concatenate([src, jnp.zeros((pad,), jnp.int32)]).reshape(NW, CH, CL)
    dst_p = jnp.concatenate([dst, pad_dst]).reshape(NW, CH, CL)
    x_p = jnp.pad(x, ((0, NP - N), (0, 0)))
    b1r = b1.reshape(1, D)
    b2r = b2.reshape(1, D)

    deg_k, agg_k = _sc_kernels()
    degp = deg_k(dst_p)
    g1 = _dense1(degp, x_p, W1)
    agg1 = agg_k(g1, src_p, dst_p)
    g2 = _dense2(degp, agg1, g1, W2, b1r)
    agg2 = agg_k(g2, src_p, dst_p)
    return _final(degp, agg2, g2, b2r)


# speedup vs baseline: 1.5137x; 1.0060x over previous
import functools

import jax
import jax.numpy as jnp
from jax import lax
from jax.experimental import pallas as pl
from jax.experimental.pallas import tpu as pltpu, tpu_sc as plsc

N = 10000
D = 128
E = 320000

NC = 2
NS = 16
NW = NC * NS
CL = 128
CH = 79
EPAD = NW * CH * CL
NP = 10240
RPS = NP // NS
RCH = RPS // CL

def _zero_block(ref, rows):
    cols = ref.shape[1] // 16

    def body(i, _):
        for k in range(cols):
            ref[i, pl.ds(k * 16, 16)] = jnp.zeros((16,), jnp.float32)
        return 0

    lax.fori_loop(0, rows, body, 0)


def _fill_rowids(idx2, base):
    def body(i, _):
        r = i // 8
        col = (i % 8) * 16
        idx2[r, pl.ds(col, 16)] = base + i * 16 + lax.iota(jnp.int32, 16)
        return 0

    lax.fori_loop(0, idx2.shape[0] * 8, body, 0)


def _deg_body(dst_hbm, out_hbm, idx_v, idx2_v, ones_v, acc_sh):
    c = lax.axis_index("c")
    s = lax.axis_index("s")
    w = c * NS + s

    _zero_block(ones_v, CL)
    _fill_rowids(idx2_v, s * RPS)

    for r in range(RCH):
        pltpu.sync_copy(ones_v, acc_sh.at[idx2_v.at[r]])
    plsc.subcore_barrier()

    one0 = jnp.where(lax.iota(jnp.int32, 16) == 0, 1.0, 0.0).astype(jnp.float32)

    def fill_ones(i, _):
        ones_v[i, pl.ds(0, 16)] = one0
        return 0

    lax.fori_loop(0, CL, fill_ones, 0)

    pltpu.sync_copy(dst_hbm.at[w], idx_v)

    def body(j, _):
        pltpu.sync_copy(ones_v, acc_sh.at[idx_v.at[j]], add=True)
        return 0

    lax.fori_loop(0, CH, body, 0)
    plsc.subcore_barrier()

    for r in range(RCH):
        pltpu.sync_copy(acc_sh.at[idx2_v.at[r]], ones_v)
        pltpu.sync_copy(ones_v, out_hbm.at[c, pl.ds(s * RPS + r * CL, CL)])


def _agg_body(g_hbm, src_hbm, dst_hbm, out_hbm,
              src_v, dst_v, idx2_v, rows0_v, sem_a, sem_b, acc_sh):
    c = lax.axis_index("c")
    s = lax.axis_index("s")
    w = c * NS + s

    _zero_block(rows0_v, CL)
    _fill_rowids(idx2_v, s * RPS)
    for r in range(RCH):
        pltpu.sync_copy(rows0_v, acc_sh.at[idx2_v.at[r]])
    plsc.subcore_barrier()

    pltpu.sync_copy(src_hbm.at[w], src_v)
    pltpu.sync_copy(dst_hbm.at[w], dst_v)

    def body(j, _):
        ga = pltpu.async_copy(
            g_hbm.at[src_v.at[2 * j]], rows0_v.at[pl.ds(0, CL // 2)], sem_a)
        gb = pltpu.async_copy(
            g_hbm.at[src_v.at[2 * j + 1]], rows0_v.at[pl.ds(CL // 2, CL // 2)],
            sem_b)
        ga.wait()
        gb.wait()
        pltpu.sync_copy(rows0_v, acc_sh.at[dst_v.at[j]], add=True)
        return 0

    lax.fori_loop(0, CH, body, 0)

    plsc.subcore_barrier()

    for r in range(RCH):
        pltpu.sync_copy(acc_sh.at[idx2_v.at[r]], rows0_v)
        pltpu.sync_copy(rows0_v, out_hbm.at[c, pl.ds(s * RPS + r * CL, CL)])


@functools.cache
def _sc_kernels():
    mesh = plsc.VectorSubcoreMesh(core_axis_name="c", subcore_axis_name="s")
    deg_k = pl.kernel(
        _deg_body,
        out_type=jax.ShapeDtypeStruct((NC, NP, D), jnp.float32),
        mesh=mesh,
        scratch_types=[
            pltpu.VMEM((CH, CL), jnp.int32),
            pltpu.VMEM((RCH, CL), jnp.int32),
            pltpu.VMEM((CL, D), jnp.float32),
            pltpu.VMEM_SHARED((NP, D), jnp.float32),
        ],
    )
    agg_k = pl.kernel(
        _agg_body,
        out_type=jax.ShapeDtypeStruct((NC, NP, D), jnp.float32),
        mesh=mesh,
        scratch_types=[
            pltpu.VMEM((2 * CH, CL // 2), jnp.int32),
            pltpu.VMEM((CH, CL), jnp.int32),
            pltpu.VMEM((RCH, CL), jnp.int32),
            pltpu.VMEM((CL, D), jnp.float32),
            pltpu.SemaphoreType.DMA,
            pltpu.SemaphoreType.DMA,
            pltpu.VMEM_SHARED((NP, D), jnp.float32),
        ],
    )
    return deg_k, agg_k


def _dis_block(degp):
    deg = degp[0, :, 0] + degp[1, :, 0]
    return lax.rsqrt(deg + 1.0)


def _dense1_body(degp_ref, x_ref, w_ref, g_ref):
    dis = _dis_block(degp_ref[...])
    h = jnp.dot(x_ref[...], w_ref[...], preferred_element_type=jnp.float32)
    g_ref[...] = h * dis[:, None]


def _dense2_body(degp_ref, aggp_ref, g_ref, w_ref, b_ref, o_ref):
    dis = _dis_block(degp_ref[...])
    agg = aggp_ref[0] + aggp_ref[1] + g_ref[...]
    act = jnp.maximum(agg * dis[:, None] + b_ref[...], 0.0)
    h = jnp.dot(act, w_ref[...], preferred_element_type=jnp.float32)
    o_ref[...] = h * dis[:, None]


def _final_body(degp_ref, aggp_ref, g_ref, b_ref, o_ref):
    dis = _dis_block(degp_ref[...])
    agg = aggp_ref[0] + aggp_ref[1] + g_ref[...]
    o_ref[...] = agg * dis[:, None] + b_ref[...]


_BR = 1024
_BRF = 1000

_dense1 = pl.pallas_call(
    _dense1_body,
    grid=(NP // _BR,),
    in_specs=[
        pl.BlockSpec((NC, _BR, D), lambda i: (0, i, 0)),
        pl.BlockSpec((_BR, D), lambda i: (i, 0)),
        pl.BlockSpec((D, D), lambda i: (0, 0)),
    ],
    out_specs=pl.BlockSpec((_BR, D), lambda i: (i, 0)),
    out_shape=jax.ShapeDtypeStruct((NP, D), jnp.float32),
)

_dense2 = pl.pallas_call(
    _dense2_body,
    grid=(NP // _BR,),
    in_specs=[
        pl.BlockSpec((NC, _BR, D), lambda i: (0, i, 0)),
        pl.BlockSpec((NC, _BR, D), lambda i: (0, i, 0)),
        pl.BlockSpec((_BR, D), lambda i: (i, 0)),
        pl.BlockSpec((D, D), lambda i: (0, 0)),
        pl.BlockSpec((1, D), lambda i: (0, 0)),
    ],
    out_specs=pl.BlockSpec((_BR, D), lambda i: (i, 0)),
    out_shape=jax.ShapeDtypeStruct((NP, D), jnp.float32),
)

_final = pl.pallas_call(
    _final_body,
    grid=(N // _BRF,),
    in_specs=[
        pl.BlockSpec((NC, _BRF, D), lambda i: (0, i, 0)),
        pl.BlockSpec((NC, _BRF, D), lambda i: (0, i, 0)),
        pl.BlockSpec((_BRF, D), lambda i: (i, 0)),
        pl.BlockSpec((1, D), lambda i: (0, 0)),
    ],
    out_specs=pl.BlockSpec((_BRF, D), lambda i: (i, 0)),
    out_shape=jax.ShapeDtypeStruct((N, D), jnp.float32),
)


def kernel(x, edge_index, W1, b1, W2, b2):
    src = edge_index[0].astype(jnp.int32)
    dst = edge_index[1].astype(jnp.int32)
    pad = EPAD - E
    pad_dst = N + (jnp.arange(pad, dtype=jnp.int32) % (NP - N))
    src_p = jnp.concatenate([src, jnp.zeros((pad,), jnp.int32)]).reshape(
        NW, 2 * CH, CL // 2)
    dst_p = jnp.concatenate([dst, pad_dst]).reshape(NW, CH, CL)
    x_p = jnp.pad(x, ((0, NP - N), (0, 0)))
    b1r = b1.reshape(1, D)
    b2r = b2.reshape(1, D)

    deg_k, agg_k = _sc_kernels()
    degp = deg_k(dst_p)
    g1 = _dense1(degp, x_p, W1)
    agg1 = agg_k(g1, src_p, dst_p)
    g2 = _dense2(degp, agg1, g1, W2, b1r)
    agg2 = agg_k(g2, src_p, dst_p)
    return _final(degp, agg2, g2, b2r)
